# Initial kernel scaffold; baseline (speedup 1.0000x reference)
import jax
import jax.numpy as jnp
from jax.experimental import pallas as pl


def kernel(x, edge_index, W1, b1, g1, be1, W2, b2, g2, be2, W3, b3, g3, be3, Wc, bc):
    raise NotImplementedError("write your pallas kernel here")


# speedup vs baseline: 24.9607x; 24.9607x over previous
import functools

import jax
import jax.numpy as jnp
from jax import lax
from jax.experimental import pallas as pl
from jax.experimental.pallas import tpu as pltpu
from jax.experimental.pallas import tpu_sc as plsc

N_NODES = 10000
D = 64
NC = 2
NS = 16
NW = NC * NS
LANE = 128
ROWS_PER_TILE = 80
E_PAD = NW * ROWS_PER_TILE * LANE
R2D = E_PAD // LANE
NPAD = 10240
SLICE = NPAD // NS

_sc_mesh = plsc.VectorSubcoreMesh(core_axis_name="c", subcore_axis_name="s")


@functools.partial(
    pl.kernel,
    out_type=jax.ShapeDtypeStruct((NC, NPAD), jnp.float32),
    mesh=_sc_mesh,
    scratch_types=[
        pltpu.VMEM((ROWS_PER_TILE, LANE), jnp.int32),
        pltpu.VMEM((LANE,), jnp.float32),
        pltpu.VMEM_SHARED((NPAD,), jnp.float32),
    ],
)
def _sc_degree(dst2d, ones_hbm, zeros1d, out, idx_v, ones_v, acc):
    c = lax.axis_index("c")
    s = lax.axis_index("s")
    wid = c * NS + s
    pltpu.sync_copy(zeros1d.at[pl.ds(s * SLICE, SLICE)],
                    acc.at[pl.ds(s * SLICE, SLICE)])
    pltpu.sync_copy(ones_hbm, ones_v)
    pltpu.sync_copy(dst2d.at[pl.ds(wid * ROWS_PER_TILE, ROWS_PER_TILE)], idx_v)
    plsc.subcore_barrier()

    def body(j, _):
        pltpu.sync_copy(ones_v, acc.at[idx_v.at[j]], add=True)
        return ()

    lax.fori_loop(0, ROWS_PER_TILE, body, ())
    plsc.subcore_barrier()
    pltpu.sync_copy(acc.at[pl.ds(s * SLICE, SLICE)],
                    out.at[c, pl.ds(s * SLICE, SLICE)])


@functools.partial(
    pl.kernel,
    out_type=jax.ShapeDtypeStruct((NC, NPAD, D), jnp.float32),
    mesh=_sc_mesh,
    scratch_types=[
        pltpu.VMEM((ROWS_PER_TILE, LANE), jnp.int32),
        pltpu.VMEM((ROWS_PER_TILE, LANE), jnp.int32),
        pltpu.VMEM((LANE, D), jnp.float32),
        pltpu.VMEM_SHARED((NPAD, D), jnp.float32),
        pltpu.SemaphoreType.DMA,
    ],
    compiler_params=pltpu.CompilerParams(use_tc_tiling_on_sc=False),
)
def _sc_propagate(h_hbm, src2d, dst2d, zeros2d, out, src_v, dst_v, rows_v,
                  acc, sem):
    c = lax.axis_index("c")
    s = lax.axis_index("s")
    wid = c * NS + s
    base = wid * ROWS_PER_TILE
    pltpu.sync_copy(zeros2d.at[pl.ds(s * SLICE, SLICE)],
                    acc.at[pl.ds(s * SLICE, SLICE)])
    pltpu.sync_copy(src2d.at[pl.ds(base, ROWS_PER_TILE)], src_v)
    pltpu.sync_copy(dst2d.at[pl.ds(base, ROWS_PER_TILE)], dst_v)
    plsc.subcore_barrier()

    def body(j, _):
        pltpu.async_copy(h_hbm.at[src_v.at[j]], rows_v, sem).wait()
        pltpu.sync_copy(rows_v, acc.at[dst_v.at[j]], add=True)
        return ()

    lax.fori_loop(0, ROWS_PER_TILE, body, ())
    plsc.subcore_barrier()
    pltpu.sync_copy(acc.at[pl.ds(s * SLICE, SLICE)],
                    out.at[c, pl.ds(s * SLICE, SLICE)])


def _tc_head_body(x_ref, w_ref, dp_ref, h_ref, dinv_ref):
    deg = dp_ref[0, :N_NODES] + dp_ref[1, :N_NODES] + 1.0
    dinv = lax.rsqrt(deg)[:, None]
    dinv_ref[...] = dinv
    h = jnp.dot(x_ref[...], w_ref[...], preferred_element_type=jnp.float32)
    h_ref[...] = h * dinv


_tc_head = pl.pallas_call(
    _tc_head_body,
    out_shape=(jax.ShapeDtypeStruct((N_NODES, D), jnp.float32),
               jax.ShapeDtypeStruct((N_NODES, 1), jnp.float32)),
)

_BN_S = float(1.0 / (1.0 + 1e-5) ** 0.5)


def _tc_mid_body(p_ref, hp_ref, dinv_ref, b_ref, g_ref, be_ref, w_ref, o_ref):
    dinv = dinv_ref[...]
    t = p_ref[0, :N_NODES, :] + p_ref[1, :N_NODES, :] + hp_ref[...]
    out1 = dinv * t + b_ref[...]
    a = jnp.maximum(out1 * (g_ref[...] * _BN_S) + be_ref[...], 0.0)
    o_ref[...] = jnp.dot(a, w_ref[...],
                         preferred_element_type=jnp.float32) * dinv


_tc_mid = pl.pallas_call(
    _tc_mid_body,
    out_shape=jax.ShapeDtypeStruct((N_NODES, D), jnp.float32),
)


def _tc_tail_body(p_ref, hp_ref, dinv_ref, b_ref, g_ref, be_ref, wc_ref,
                  bc_ref, o_ref):
    dinv = dinv_ref[...]
    t = p_ref[0, :N_NODES, :] + p_ref[1, :N_NODES, :] + hp_ref[...]
    out1 = dinv * t + b_ref[...]
    a = jnp.maximum(out1 * (g_ref[...] * _BN_S) + be_ref[...], 0.0)
    logits = jnp.dot(a, wc_ref[...],
                     preferred_element_type=jnp.float32) + bc_ref[...]
    m = jnp.max(logits, axis=1, keepdims=True)
    lse = jnp.log(jnp.sum(jnp.exp(logits - m), axis=1, keepdims=True)) + m
    o_ref[...] = logits - lse


_tc_tail = pl.pallas_call(
    _tc_tail_body,
    out_shape=jax.ShapeDtypeStruct((N_NODES, 2), jnp.float32),
)


def kernel(x, edge_index, W1, b1, g1, be1, W2, b2, g2, be2, W3, b3, g3, be3,
           Wc, bc):
    src = edge_index[0].astype(jnp.int32)
    dst = edge_index[1].astype(jnp.int32)
    pad_n = E_PAD - src.shape[0]
    pad_i = jnp.arange(pad_n, dtype=jnp.int32)
    pad_src = (pad_i * 97) % N_NODES
    pad_dst = N_NODES + pad_i % (NPAD - N_NODES)
    src2d = jnp.concatenate([src, pad_src]).reshape(R2D, LANE)
    dst2d = jnp.concatenate([dst, pad_dst]).reshape(R2D, LANE)

    ones_lane = jnp.ones((LANE,), jnp.float32)
    zeros1d = jnp.zeros((NPAD,), jnp.float32)
    zeros2d = jnp.zeros((NPAD, D), jnp.float32)

    deg_parts = _sc_degree(dst2d, ones_lane, zeros1d)
    h1p, dinv = _tc_head(x, W1, deg_parts)

    p1 = _sc_propagate(h1p, src2d, dst2d, zeros2d)
    h2p = _tc_mid(p1, h1p, dinv, b1.reshape(1, D), g1.reshape(1, D),
                  be1.reshape(1, D), W2)
    p2 = _sc_propagate(h2p, src2d, dst2d, zeros2d)
    h3p = _tc_mid(p2, h2p, dinv, b2.reshape(1, D), g2.reshape(1, D),
                  be2.reshape(1, D), W3)
    p3 = _sc_propagate(h3p, src2d, dst2d, zeros2d)
    return _tc_tail(p3, h3p, dinv, b3.reshape(1, D), g3.reshape(1, D),
                    be3.reshape(1, D), Wc, bc.reshape(1, 2))


# speedup vs baseline: 39.4879x; 1.5820x over previous
import functools

import jax
import jax.numpy as jnp
from jax import lax
from jax.experimental import pallas as pl
from jax.experimental.pallas import tpu as pltpu
from jax.experimental.pallas import tpu_sc as plsc

N_NODES = 10000
D = 64
NC = 2
NS = 16
NW = NC * NS
LANE = 128
ROWS_PER_TILE = 80
E_PAD = NW * ROWS_PER_TILE * LANE
R2D = E_PAD // LANE
NPAD = 10240
SLICE = NPAD // NS

_sc_mesh = plsc.VectorSubcoreMesh(core_axis_name="c", subcore_axis_name="s")


@functools.partial(
    pl.kernel,
    out_type=jax.ShapeDtypeStruct((NC, NPAD), jnp.float32),
    mesh=_sc_mesh,
    scratch_types=[
        pltpu.VMEM((ROWS_PER_TILE, LANE), jnp.int32),
        pltpu.VMEM((LANE,), jnp.float32),
        pltpu.VMEM_SHARED((NPAD,), jnp.float32),
    ],
)
def _sc_degree(dst2d, ones_hbm, zeros1d, out, idx_v, ones_v, acc):
    c = lax.axis_index("c")
    s = lax.axis_index("s")
    wid = c * NS + s
    pltpu.sync_copy(zeros1d.at[pl.ds(s * SLICE, SLICE)],
                    acc.at[pl.ds(s * SLICE, SLICE)])
    pltpu.sync_copy(ones_hbm, ones_v)
    pltpu.sync_copy(dst2d.at[pl.ds(wid * ROWS_PER_TILE, ROWS_PER_TILE)], idx_v)
    plsc.subcore_barrier()

    def body(j, _):
        pltpu.sync_copy(ones_v, acc.at[idx_v.at[j]], add=True)
        return ()

    lax.fori_loop(0, ROWS_PER_TILE, body, ())
    plsc.subcore_barrier()
    pltpu.sync_copy(acc.at[pl.ds(s * SLICE, SLICE)],
                    out.at[c, pl.ds(s * SLICE, SLICE)])


NBUF = 8
N_ITERS = ROWS_PER_TILE // NBUF


@functools.partial(
    pl.kernel,
    out_type=jax.ShapeDtypeStruct((NC, NPAD, D), jnp.float32),
    mesh=_sc_mesh,
    scratch_types=(
        [pltpu.VMEM((ROWS_PER_TILE, LANE), jnp.int32),
         pltpu.VMEM((ROWS_PER_TILE, LANE), jnp.int32)]
        + [pltpu.VMEM((LANE, D), jnp.float32) for _ in range(NBUF)]
        + [pltpu.VMEM_SHARED((NPAD, D), jnp.float32)]
        + [pltpu.SemaphoreType.DMA for _ in range(2 * NBUF)]
    ),
    compiler_params=pltpu.CompilerParams(use_tc_tiling_on_sc=False),
)
def _sc_propagate(h_hbm, src2d, dst2d, zeros2d, out, src_v, dst_v, *rest):
    bufs = rest[:NBUF]
    acc = rest[NBUF]
    semg = rest[NBUF + 1:NBUF + 1 + NBUF]
    sems = rest[NBUF + 1 + NBUF:]
    c = lax.axis_index("c")
    s = lax.axis_index("s")
    wid = c * NS + s
    base = wid * ROWS_PER_TILE
    pltpu.sync_copy(zeros2d.at[pl.ds(s * SLICE, SLICE)],
                    acc.at[pl.ds(s * SLICE, SLICE)])
    pltpu.sync_copy(src2d.at[pl.ds(base, ROWS_PER_TILE)], src_v)
    pltpu.sync_copy(dst2d.at[pl.ds(base, ROWS_PER_TILE)], dst_v)
    plsc.subcore_barrier()

    for k in range(NBUF):
        pltpu.async_copy(h_hbm.at[src_v.at[k]], bufs[k], semg[k])

    def body(i, _):
        for k in range(NBUF):
            w = i * NBUF + k
            pltpu.make_async_copy(h_hbm.at[src_v.at[w]], bufs[k],
                                  semg[k]).wait()
            pltpu.async_copy(bufs[k], acc.at[dst_v.at[w]], sems[k], add=True)

        @pl.when(i < N_ITERS - 1)
        def _():
            for k in range(NBUF):
                w = i * NBUF + k
                pltpu.make_async_copy(bufs[k], acc.at[dst_v.at[w]],
                                      sems[k]).wait()
                pltpu.async_copy(h_hbm.at[src_v.at[w + NBUF]], bufs[k],
                                 semg[k])
        return ()

    lax.fori_loop(0, N_ITERS, body, ())
    for k in range(NBUF):
        pltpu.make_async_copy(bufs[k], acc.at[dst_v.at[0]], sems[k]).wait()
    plsc.subcore_barrier()
    pltpu.sync_copy(acc.at[pl.ds(s * SLICE, SLICE)],
                    out.at[c, pl.ds(s * SLICE, SLICE)])


def _tc_head_body(x_ref, w_ref, dp_ref, h_ref, dinv_ref):
    deg = dp_ref[0, :N_NODES] + dp_ref[1, :N_NODES] + 1.0
    dinv = lax.rsqrt(deg)[:, None]
    dinv_ref[...] = dinv
    h = jnp.dot(x_ref[...], w_ref[...], preferred_element_type=jnp.float32)
    h_ref[...] = h * dinv


_tc_head = pl.pallas_call(
    _tc_head_body,
    out_shape=(jax.ShapeDtypeStruct((N_NODES, D), jnp.float32),
               jax.ShapeDtypeStruct((N_NODES, 1), jnp.float32)),
)

_BN_S = float(1.0 / (1.0 + 1e-5) ** 0.5)


def _tc_mid_body(p_ref, hp_ref, dinv_ref, b_ref, g_ref, be_ref, w_ref, o_ref):
    dinv = dinv_ref[...]
    t = p_ref[0, :N_NODES, :] + p_ref[1, :N_NODES, :] + hp_ref[...]
    out1 = dinv * t + b_ref[...]
    a = jnp.maximum(out1 * (g_ref[...] * _BN_S) + be_ref[...], 0.0)
    o_ref[...] = jnp.dot(a, w_ref[...],
                         preferred_element_type=jnp.float32) * dinv


_tc_mid = pl.pallas_call(
    _tc_mid_body,
    out_shape=jax.ShapeDtypeStruct((N_NODES, D), jnp.float32),
)


def _tc_tail_body(p_ref, hp_ref, dinv_ref, b_ref, g_ref, be_ref, wc_ref,
                  bc_ref, o_ref):
    dinv = dinv_ref[...]
    t = p_ref[0, :N_NODES, :] + p_ref[1, :N_NODES, :] + hp_ref[...]
    out1 = dinv * t + b_ref[...]
    a = jnp.maximum(out1 * (g_ref[...] * _BN_S) + be_ref[...], 0.0)
    logits = jnp.dot(a, wc_ref[...],
                     preferred_element_type=jnp.float32) + bc_ref[...]
    m = jnp.max(logits, axis=1, keepdims=True)
    lse = jnp.log(jnp.sum(jnp.exp(logits - m), axis=1, keepdims=True)) + m
    o_ref[...] = logits - lse


_tc_tail = pl.pallas_call(
    _tc_tail_body,
    out_shape=jax.ShapeDtypeStruct((N_NODES, 2), jnp.float32),
)


def kernel(x, edge_index, W1, b1, g1, be1, W2, b2, g2, be2, W3, b3, g3, be3,
           Wc, bc):
    src = edge_index[0].astype(jnp.int32)
    dst = edge_index[1].astype(jnp.int32)
    pad_n = E_PAD - src.shape[0]
    pad_i = jnp.arange(pad_n, dtype=jnp.int32)
    pad_src = (pad_i * 97) % N_NODES
    pad_dst = N_NODES + pad_i % (NPAD - N_NODES)
    src2d = jnp.concatenate([src, pad_src]).reshape(R2D, LANE)
    dst2d = jnp.concatenate([dst, pad_dst]).reshape(R2D, LANE)

    ones_lane = jnp.ones((LANE,), jnp.float32)
    zeros1d = jnp.zeros((NPAD,), jnp.float32)
    zeros2d = jnp.zeros((NPAD, D), jnp.float32)

    deg_parts = _sc_degree(dst2d, ones_lane, zeros1d)
    h1p, dinv = _tc_head(x, W1, deg_parts)

    p1 = _sc_propagate(h1p, src2d, dst2d, zeros2d)
    h2p = _tc_mid(p1, h1p, dinv, b1.reshape(1, D), g1.reshape(1, D),
                  be1.reshape(1, D), W2)
    p2 = _sc_propagate(h2p, src2d, dst2d, zeros2d)
    h3p = _tc_mid(p2, h2p, dinv, b2.reshape(1, D), g2.reshape(1, D),
                  be2.reshape(1, D), W3)
    p3 = _sc_propagate(h3p, src2d, dst2d, zeros2d)
    return _tc_tail(p3, h3p, dinv, b3.reshape(1, D), g3.reshape(1, D),
                    be3.reshape(1, D), Wc, bc.reshape(1, 2))


# speedup vs baseline: 40.1648x; 1.0171x over previous
import functools

import jax
import jax.numpy as jnp
from jax import lax
from jax.experimental import pallas as pl
from jax.experimental.pallas import tpu as pltpu
from jax.experimental.pallas import tpu_sc as plsc

N_NODES = 10000
D = 64
NC = 2
NS = 16
NW = NC * NS
LANE = 128
ROWS_PER_TILE = 80
E_PAD = NW * ROWS_PER_TILE * LANE
R2D = E_PAD // LANE
NPAD = 10240
SLICE = NPAD // NS

_sc_mesh = plsc.VectorSubcoreMesh(core_axis_name="c", subcore_axis_name="s")


@functools.partial(
    pl.kernel,
    out_type=jax.ShapeDtypeStruct((NC, NPAD), jnp.float32),
    mesh=_sc_mesh,
    scratch_types=[
        pltpu.VMEM((ROWS_PER_TILE, LANE), jnp.int32),
        pltpu.VMEM((LANE,), jnp.float32),
        pltpu.VMEM_SHARED((NPAD,), jnp.float32),
        pltpu.SemaphoreType.DMA,
    ],
)
def _sc_degree(dst2d, ones_hbm, zeros1d, out, idx_v, ones_v, acc, semd):
    c = lax.axis_index("c")
    s = lax.axis_index("s")
    wid = c * NS + s
    pltpu.sync_copy(zeros1d.at[pl.ds(s * SLICE, SLICE)],
                    acc.at[pl.ds(s * SLICE, SLICE)])
    pltpu.sync_copy(ones_hbm, ones_v)
    pltpu.sync_copy(dst2d.at[pl.ds(wid * ROWS_PER_TILE, ROWS_PER_TILE)], idx_v)
    plsc.subcore_barrier()

    def body(j, _):
        pltpu.async_copy(ones_v, acc.at[idx_v.at[j]], semd, add=True)
        return ()

    lax.fori_loop(0, ROWS_PER_TILE, body, ())

    def drain(j, _):
        pltpu.make_async_copy(ones_v, acc.at[idx_v.at[0]], semd).wait()
        return ()

    lax.fori_loop(0, ROWS_PER_TILE, drain, ())
    plsc.subcore_barrier()
    pltpu.sync_copy(acc.at[pl.ds(s * SLICE, SLICE)],
                    out.at[c, pl.ds(s * SLICE, SLICE)])


NBUF = 8
N_ITERS = ROWS_PER_TILE // NBUF


@functools.partial(
    pl.kernel,
    out_type=jax.ShapeDtypeStruct((NC, NPAD, D), jnp.float32),
    mesh=_sc_mesh,
    scratch_types=(
        [pltpu.VMEM((ROWS_PER_TILE, LANE), jnp.int32),
         pltpu.VMEM((ROWS_PER_TILE, LANE), jnp.int32)]
        + [pltpu.VMEM((LANE, D), jnp.float32) for _ in range(NBUF)]
        + [pltpu.VMEM_SHARED((NPAD, D), jnp.float32)]
        + [pltpu.SemaphoreType.DMA for _ in range(2 * NBUF)]
    ),
    compiler_params=pltpu.CompilerParams(use_tc_tiling_on_sc=False),
)
def _sc_propagate(h_hbm, src2d, dst2d, zeros2d, out, src_v, dst_v, *rest):
    bufs = rest[:NBUF]
    acc = rest[NBUF]
    semg = rest[NBUF + 1:NBUF + 1 + NBUF]
    sems = rest[NBUF + 1 + NBUF:]
    c = lax.axis_index("c")
    s = lax.axis_index("s")
    wid = c * NS + s
    base = wid * ROWS_PER_TILE
    pltpu.sync_copy(zeros2d.at[pl.ds(s * SLICE, SLICE)],
                    acc.at[pl.ds(s * SLICE, SLICE)])
    pltpu.sync_copy(src2d.at[pl.ds(base, ROWS_PER_TILE)], src_v)
    pltpu.sync_copy(dst2d.at[pl.ds(base, ROWS_PER_TILE)], dst_v)
    plsc.subcore_barrier()

    for k in range(NBUF):
        pltpu.async_copy(h_hbm.at[src_v.at[k]], bufs[k], semg[k])

    def body(i, _):
        for k in range(NBUF):
            w = i * NBUF + k
            pltpu.make_async_copy(h_hbm.at[src_v.at[w]], bufs[k],
                                  semg[k]).wait()
            pltpu.async_copy(bufs[k], acc.at[dst_v.at[w]], sems[k], add=True)

        @pl.when(i < N_ITERS - 1)
        def _():
            for k in range(NBUF):
                w = i * NBUF + k
                pltpu.make_async_copy(bufs[k], acc.at[dst_v.at[w]],
                                      sems[k]).wait()
                pltpu.async_copy(h_hbm.at[src_v.at[w + NBUF]], bufs[k],
                                 semg[k])
        return ()

    lax.fori_loop(0, N_ITERS, body, ())
    for k in range(NBUF):
        pltpu.make_async_copy(bufs[k], acc.at[dst_v.at[0]], sems[k]).wait()
    plsc.subcore_barrier()
    pltpu.sync_copy(acc.at[pl.ds(s * SLICE, SLICE)],
                    out.at[c, pl.ds(s * SLICE, SLICE)])


def _tc_head_body(x_ref, w_ref, dp_ref, h_ref, dinv_ref):
    deg = dp_ref[0, :N_NODES] + dp_ref[1, :N_NODES] + 1.0
    dinv = lax.rsqrt(deg)[:, None]
    dinv_ref[...] = dinv
    h = jnp.dot(x_ref[...], w_ref[...], preferred_element_type=jnp.float32)
    h_ref[...] = h * dinv


_tc_head = pl.pallas_call(
    _tc_head_body,
    out_shape=(jax.ShapeDtypeStruct((N_NODES, D), jnp.float32),
               jax.ShapeDtypeStruct((N_NODES, 1), jnp.float32)),
)

_BN_S = float(1.0 / (1.0 + 1e-5) ** 0.5)


def _tc_mid_body(p_ref, hp_ref, dinv_ref, b_ref, g_ref, be_ref, w_ref, o_ref):
    dinv = dinv_ref[...]
    t = p_ref[0, :N_NODES, :] + p_ref[1, :N_NODES, :] + hp_ref[...]
    out1 = dinv * t + b_ref[...]
    a = jnp.maximum(out1 * (g_ref[...] * _BN_S) + be_ref[...], 0.0)
    o_ref[...] = jnp.dot(a, w_ref[...],
                         preferred_element_type=jnp.float32) * dinv


_tc_mid = pl.pallas_call(
    _tc_mid_body,
    out_shape=jax.ShapeDtypeStruct((N_NODES, D), jnp.float32),
)


def _tc_tail_body(p_ref, hp_ref, dinv_ref, b_ref, g_ref, be_ref, wc_ref,
                  bc_ref, o_ref):
    dinv = dinv_ref[...]
    t = p_ref[0, :N_NODES, :] + p_ref[1, :N_NODES, :] + hp_ref[...]
    out1 = dinv * t + b_ref[...]
    a = jnp.maximum(out1 * (g_ref[...] * _BN_S) + be_ref[...], 0.0)
    logits = jnp.dot(a, wc_ref[...],
                     preferred_element_type=jnp.float32) + bc_ref[...]
    m = jnp.max(logits, axis=1, keepdims=True)
    lse = jnp.log(jnp.sum(jnp.exp(logits - m), axis=1, keepdims=True)) + m
    o_ref[...] = logits - lse


_tc_tail = pl.pallas_call(
    _tc_tail_body,
    out_shape=jax.ShapeDtypeStruct((N_NODES, 2), jnp.float32),
)


def kernel(x, edge_index, W1, b1, g1, be1, W2, b2, g2, be2, W3, b3, g3, be3,
           Wc, bc):
    src = edge_index[0].astype(jnp.int32)
    dst = edge_index[1].astype(jnp.int32)
    pad_n = E_PAD - src.shape[0]
    pad_i = jnp.arange(pad_n, dtype=jnp.int32)
    pad_src = (pad_i * 97) % N_NODES
    pad_dst = N_NODES + pad_i % (NPAD - N_NODES)
    src2d = jnp.concatenate([src, pad_src]).reshape(R2D, LANE)
    dst2d = jnp.concatenate([dst, pad_dst]).reshape(R2D, LANE)

    ones_lane = jnp.ones((LANE,), jnp.float32)
    zeros1d = jnp.zeros((NPAD,), jnp.float32)
    zeros2d = jnp.zeros((NPAD, D), jnp.float32)

    deg_parts = _sc_degree(dst2d, ones_lane, zeros1d)
    h1p, dinv = _tc_head(x, W1, deg_parts)

    p1 = _sc_propagate(h1p, src2d, dst2d, zeros2d)
    h2p = _tc_mid(p1, h1p, dinv, b1.reshape(1, D), g1.reshape(1, D),
                  be1.reshape(1, D), W2)
    p2 = _sc_propagate(h2p, src2d, dst2d, zeros2d)
    h3p = _tc_mid(p2, h2p, dinv, b2.reshape(1, D), g2.reshape(1, D),
                  be2.reshape(1, D), W3)
    p3 = _sc_propagate(h3p, src2d, dst2d, zeros2d)
    return _tc_tail(p3, h3p, dinv, b3.reshape(1, D), g3.reshape(1, D),
                    be3.reshape(1, D), Wc, bc.reshape(1, 2))


# speedup vs baseline: 41.2006x; 1.0258x over previous
import functools

import jax
import jax.numpy as jnp
from jax import lax
from jax.experimental import pallas as pl
from jax.experimental.pallas import tpu as pltpu
from jax.experimental.pallas import tpu_sc as plsc

N_NODES = 10000
D = 64
NC = 2
NS = 16
NW = NC * NS
LANE = 128
ROWS_PER_TILE = 80
E_PAD = NW * ROWS_PER_TILE * LANE
R2D = E_PAD // LANE
NPAD = 10240
SLICE = NPAD // NS

_sc_mesh = plsc.VectorSubcoreMesh(core_axis_name="c", subcore_axis_name="s")


@functools.partial(
    pl.kernel,
    out_type=jax.ShapeDtypeStruct((NC, NPAD), jnp.float32),
    mesh=_sc_mesh,
    scratch_types=[
        pltpu.VMEM((ROWS_PER_TILE, LANE), jnp.int32),
        pltpu.VMEM((LANE,), jnp.float32),
        pltpu.VMEM_SHARED((NPAD,), jnp.float32),
        pltpu.SemaphoreType.DMA,
    ],
)
def _sc_degree(dst2d, ones_hbm, zeros1d, out, idx_v, ones_v, acc, semd):
    c = lax.axis_index("c")
    s = lax.axis_index("s")
    wid = c * NS + s
    pltpu.sync_copy(zeros1d.at[pl.ds(s * SLICE, SLICE)],
                    acc.at[pl.ds(s * SLICE, SLICE)])
    pltpu.sync_copy(ones_hbm, ones_v)
    pltpu.sync_copy(dst2d.at[pl.ds(wid * ROWS_PER_TILE, ROWS_PER_TILE)], idx_v)
    plsc.subcore_barrier()

    def body(j, _):
        pltpu.async_copy(ones_v, acc.at[idx_v.at[j]], semd, add=True)
        return ()

    lax.fori_loop(0, ROWS_PER_TILE, body, ())

    def drain(j, _):
        pltpu.make_async_copy(ones_v, acc.at[idx_v.at[0]], semd).wait()
        return ()

    lax.fori_loop(0, ROWS_PER_TILE, drain, ())
    plsc.subcore_barrier()
    pltpu.sync_copy(acc.at[pl.ds(s * SLICE, SLICE)],
                    out.at[c, pl.ds(s * SLICE, SLICE)])


NBUF = 8
N_ITERS = ROWS_PER_TILE // NBUF


@functools.partial(
    pl.kernel,
    out_type=jax.ShapeDtypeStruct((NC, NPAD, D), jnp.float32),
    mesh=_sc_mesh,
    scratch_types=(
        [pltpu.VMEM((ROWS_PER_TILE, LANE), jnp.int32),
         pltpu.VMEM((ROWS_PER_TILE, LANE), jnp.int32)]
        + [pltpu.VMEM((LANE, D), jnp.float32) for _ in range(NBUF)]
        + [pltpu.VMEM_SHARED((NPAD, D), jnp.float32)]
        + [pltpu.SemaphoreType.DMA for _ in range(2 * NBUF + 2)]
    ),
    compiler_params=pltpu.CompilerParams(use_tc_tiling_on_sc=False),
)
def _sc_propagate(h_hbm, src2d, dst2d, zeros2d, out, src_v, dst_v, *rest):
    bufs = rest[:NBUF]
    acc = rest[NBUF]
    semg = rest[NBUF + 1:NBUF + 1 + NBUF]
    sems = rest[NBUF + 1 + NBUF:NBUF + 1 + 2 * NBUF]
    semz, semi = rest[NBUF + 1 + 2 * NBUF:]
    c = lax.axis_index("c")
    s = lax.axis_index("s")
    wid = c * NS + s
    base = wid * ROWS_PER_TILE
    pltpu.async_copy(zeros2d.at[pl.ds(s * SLICE, SLICE)],
                     acc.at[pl.ds(s * SLICE, SLICE)], semz)
    pltpu.async_copy(src2d.at[pl.ds(base, ROWS_PER_TILE)], src_v, semi)
    pltpu.async_copy(dst2d.at[pl.ds(base, ROWS_PER_TILE)], dst_v, semi)
    pltpu.make_async_copy(src2d.at[pl.ds(base, ROWS_PER_TILE)], src_v,
                          semi).wait()
    pltpu.make_async_copy(dst2d.at[pl.ds(base, ROWS_PER_TILE)], dst_v,
                          semi).wait()

    for k in range(NBUF):
        pltpu.async_copy(h_hbm.at[src_v.at[k]], bufs[k], semg[k])

    pltpu.make_async_copy(zeros2d.at[pl.ds(s * SLICE, SLICE)],
                          acc.at[pl.ds(s * SLICE, SLICE)], semz).wait()
    plsc.subcore_barrier()

    def body(i, _):
        for k in range(NBUF):
            w = i * NBUF + k
            pltpu.make_async_copy(h_hbm.at[src_v.at[w]], bufs[k],
                                  semg[k]).wait()
            pltpu.async_copy(bufs[k], acc.at[dst_v.at[w]], sems[k], add=True)

        @pl.when(i < N_ITERS - 1)
        def _():
            for k in range(NBUF):
                w = i * NBUF + k
                pltpu.make_async_copy(bufs[k], acc.at[dst_v.at[w]],
                                      sems[k]).wait()
                pltpu.async_copy(h_hbm.at[src_v.at[w + NBUF]], bufs[k],
                                 semg[k])
        return ()

    lax.fori_loop(0, N_ITERS, body, ())
    for k in range(NBUF):
        pltpu.make_async_copy(bufs[k], acc.at[dst_v.at[0]], sems[k]).wait()
    plsc.subcore_barrier()
    pltpu.sync_copy(acc.at[pl.ds(s * SLICE, SLICE)],
                    out.at[c, pl.ds(s * SLICE, SLICE)])


def _tc_head_body(x_ref, w_ref, dp_ref, h_ref, dinv_ref):
    deg = dp_ref[0, :N_NODES] + dp_ref[1, :N_NODES] + 1.0
    dinv = lax.rsqrt(deg)[:, None]
    dinv_ref[...] = dinv
    h = jnp.dot(x_ref[...], w_ref[...], preferred_element_type=jnp.float32)
    h_ref[...] = h * dinv


_tc_head = pl.pallas_call(
    _tc_head_body,
    out_shape=(jax.ShapeDtypeStruct((N_NODES, D), jnp.float32),
               jax.ShapeDtypeStruct((N_NODES, 1), jnp.float32)),
)

_BN_S = float(1.0 / (1.0 + 1e-5) ** 0.5)


def _tc_mid_body(p_ref, hp_ref, dinv_ref, b_ref, g_ref, be_ref, w_ref, o_ref):
    dinv = dinv_ref[...]
    t = p_ref[0, :N_NODES, :] + p_ref[1, :N_NODES, :] + hp_ref[...]
    out1 = dinv * t + b_ref[...]
    a = jnp.maximum(out1 * (g_ref[...] * _BN_S) + be_ref[...], 0.0)
    o_ref[...] = jnp.dot(a, w_ref[...],
                         preferred_element_type=jnp.float32) * dinv


_tc_mid = pl.pallas_call(
    _tc_mid_body,
    out_shape=jax.ShapeDtypeStruct((N_NODES, D), jnp.float32),
)


def _tc_tail_body(p_ref, hp_ref, dinv_ref, b_ref, g_ref, be_ref, wc_ref,
                  bc_ref, o_ref):
    dinv = dinv_ref[...]
    t = p_ref[0, :N_NODES, :] + p_ref[1, :N_NODES, :] + hp_ref[...]
    out1 = dinv * t + b_ref[...]
    a = jnp.maximum(out1 * (g_ref[...] * _BN_S) + be_ref[...], 0.0)
    logits = jnp.dot(a, wc_ref[...],
                     preferred_element_type=jnp.float32) + bc_ref[...]
    m = jnp.max(logits, axis=1, keepdims=True)
    lse = jnp.log(jnp.sum(jnp.exp(logits - m), axis=1, keepdims=True)) + m
    o_ref[...] = logits - lse


_tc_tail = pl.pallas_call(
    _tc_tail_body,
    out_shape=jax.ShapeDtypeStruct((N_NODES, 2), jnp.float32),
)


def kernel(x, edge_index, W1, b1, g1, be1, W2, b2, g2, be2, W3, b3, g3, be3,
           Wc, bc):
    src = edge_index[0].astype(jnp.int32)
    dst = edge_index[1].astype(jnp.int32)
    pad_n = E_PAD - src.shape[0]
    pad_i = jnp.arange(pad_n, dtype=jnp.int32)
    pad_src = (pad_i * 97) % N_NODES
    pad_dst = N_NODES + pad_i % (NPAD - N_NODES)
    src2d = jnp.concatenate([src, pad_src]).reshape(R2D, LANE)
    dst2d = jnp.concatenate([dst, pad_dst]).reshape(R2D, LANE)

    ones_lane = jnp.ones((LANE,), jnp.float32)
    zeros1d = jnp.zeros((NPAD,), jnp.float32)
    zeros2d = jnp.zeros((NPAD, D), jnp.float32)

    deg_parts = _sc_degree(dst2d, ones_lane, zeros1d)
    h1p, dinv = _tc_head(x, W1, deg_parts)

    p1 = _sc_propagate(h1p, src2d, dst2d, zeros2d)
    h2p = _tc_mid(p1, h1p, dinv, b1.reshape(1, D), g1.reshape(1, D),
                  be1.reshape(1, D), W2)
    p2 = _sc_propagate(h2p, src2d, dst2d, zeros2d)
    h3p = _tc_mid(p2, h2p, dinv, b2.reshape(1, D), g2.reshape(1, D),
                  be2.reshape(1, D), W3)
    p3 = _sc_propagate(h3p, src2d, dst2d, zeros2d)
    return _tc_tail(p3, h3p, dinv, b3.reshape(1, D), g3.reshape(1, D),
                    be3.reshape(1, D), Wc, bc.reshape(1, 2))
